# baseline (speedup 1.0000x reference)
import functools

import jax
import jax.numpy as jnp
import numpy as np
from jax import lax
from jax.experimental import pallas as pl
from jax.experimental.pallas import tpu as pltpu
from jax.experimental.pallas import tpu_sc as plsc

NCORE = 2
NSUB = 16
NW = NCORE * NSUB
LANES = 16
CHUNK = 64
FDIM = 128
FB = FDIM // LANES
FB2 = FB // 2

_PERM = np.concatenate(
    [np.concatenate([np.arange(32 * k, 32 * k + 32, 2),
                     np.arange(32 * k + 1, 32 * k + 32, 2)])
     for k in range(FB2)])
_INV = np.argsort(_PERM)


def _mm2_body(x_ref, wa_ref, wb_ref, o_ref):
    xb = x_ref[...]
    o_ref[0] = jnp.dot(xb, wa_ref[...], preferred_element_type=jnp.float32)
    o_ref[1] = jnp.dot(xb, wb_ref[...], preferred_element_type=jnp.float32)


def _mm2(x, wa, wb):
    npad = x.shape[0]
    blk = 1024
    return pl.pallas_call(
        _mm2_body,
        grid=(npad // blk,),
        in_specs=[
            pl.BlockSpec((blk, FDIM), lambda i: (i, 0)),
            pl.BlockSpec((FDIM, FDIM), lambda i: (0, 0)),
            pl.BlockSpec((FDIM, FDIM), lambda i: (0, 0)),
        ],
        out_specs=pl.BlockSpec((2, blk, FDIM), lambda i: (0, i, 0)),
        out_shape=jax.ShapeDtypeStruct((2, npad, FDIM), jnp.float32),
    )(x, wa, wb)


def _edge_pass_body(nchunks, npad, tab_hbm, att_hbm, idx_hbm,
                    num_out, den_out,
                    idxc0, idxc1, rows0, rows1, scaled, didxs,
                    exbuf, attbuf, acc_sh, den_sh,
                    semg0, semg1, semi0, semi1):
    c = lax.axis_index("c")
    s = lax.axis_index("s")
    w = s * NCORE + c
    rows_per_sub = npad // NSUB
    lane = lax.iota(jnp.int32, LANES)
    zero16 = jnp.zeros((LANES,), jnp.float32)

    def zrow(e, _):
        for k in range(FB):
            scaled[e, pl.ds(k * LANES, LANES)] = zero16
        return 0
    lax.fori_loop(0, CHUNK, zrow, 0)
    for k in range(CHUNK // LANES):
        exbuf[pl.ds(k * LANES, LANES)] = zero16
    base = s * rows_per_sub
    for t in range(rows_per_sub // CHUNK):
        pltpu.sync_copy(scaled, acc_sh.at[pl.ds(base + t * CHUNK, CHUNK)])
        pltpu.sync_copy(exbuf, den_sh.at[pl.ds(base + t * CHUNK, CHUNK)])
    plsc.subcore_barrier()

    pltpu.sync_copy(att_hbm, attbuf)
    attv = [attbuf[pl.ds(k * LANES, LANES)] for k in range(FB)]

    def start_idx(j, idxc, semi):
        pltpu.async_copy(idx_hbm.at[w].at[j], idxc, semi)

    def wait_idx(idxc, semi):
        pltpu.make_async_copy(idx_hbm.at[w].at[0], idxc, semi).wait()

    def start_gather(idxc, rows, semg):
        pltpu.async_copy(tab_hbm.at[idxc.at[0]], rows, semg)

    def wait_gather(rows, semg):
        pltpu.make_async_copy(tab_hbm.at[pl.ds(0, 2 * CHUNK)], rows,
                              semg).wait()

    def compute_scatter(rows):
        ilv = plsc.PackFormat.INTERLEAVED

        @plsc.parallel_loop(0, CHUNK, unroll=2)
        def ebody(e):
            acc = zero16
            for k in range(FB2):
                wl = plsc.bitcast(rows[e, pl.ds(k * LANES, LANES)],
                                  jnp.bfloat16)
                wr = plsc.bitcast(rows[CHUNK + e, pl.ds(k * LANES, LANES)],
                                  jnp.bfloat16)
                a0, a1 = plsc.unpack(wl, format=ilv)
                b0, b1 = plsc.unpack(wr, format=ilv)
                z0 = a0 + b0
                z1 = a1 + b1
                acc = (acc + attv[2 * k] * jnp.maximum(z0, 0.2 * z0) +
                       attv[2 * k + 1] * jnp.maximum(z1, 0.2 * z1))
            rows[CHUNK + e, pl.ds(0, LANES)] = plsc.bitcast(acc, jnp.int32)

        @plsc.parallel_loop(0, CHUNK // LANES, unroll=2)
        def bbody(g):
            elan = CHUNK + g * LANES + lane
            lsum = plsc.bitcast(
                plsc.load_gather(rows, [elan, jnp.zeros((LANES,), jnp.int32)]),
                jnp.float32)
            for i in range(1, LANES):
                lsum = lsum + plsc.bitcast(
                    plsc.load_gather(rows,
                                     [elan, jnp.full((LANES,), i, jnp.int32)]),
                    jnp.float32)
            exbuf[pl.ds(g * LANES, LANES)] = jnp.exp(lsum)

        @plsc.parallel_loop(0, CHUNK // LANES)
        def cbody(g):
            ev = exbuf[pl.ds(g * LANES, LANES)]
            for i in range(LANES):
                exv = jnp.full((LANES,), ev[i], jnp.float32)
                e = g * LANES + i
                for k in range(FB2):
                    wl = plsc.bitcast(rows[e, pl.ds(k * LANES, LANES)],
                                      jnp.bfloat16)
                    a0, a1 = plsc.unpack(wl, format=ilv)
                    scaled[e, pl.ds(2 * k * LANES, LANES)] = exv * a0
                    scaled[e, pl.ds((2 * k + 1) * LANES, LANES)] = exv * a1

        pltpu.sync_copy(scaled, acc_sh.at[didxs.at[0]], add=True)
        pltpu.sync_copy(exbuf, den_sh.at[didxs.at[0]], add=True)

    def copy_didx(idxc):
        for k in range(CHUNK // LANES):
            didxs[0, pl.ds(k * LANES, LANES)] = idxc[1, pl.ds(k * LANES,
                                                              LANES)]

    npairs = nchunks // 2
    start_idx(0, idxc0, semi0)
    start_idx(1, idxc1, semi1)
    wait_idx(idxc0, semi0)
    start_gather(idxc0, rows0, semg0)

    def pair(p, _):
        not_last = p < npairs - 1
        wait_idx(idxc1, semi1)
        start_gather(idxc1, rows1, semg1)
        wait_gather(rows0, semg0)
        copy_didx(idxc0)

        @pl.when(not_last)
        def _():
            start_idx(2 * p + 2, idxc0, semi0)
        compute_scatter(rows0)

        @pl.when(not_last)
        def _():
            wait_idx(idxc0, semi0)
            start_gather(idxc0, rows0, semg0)
        wait_gather(rows1, semg1)
        copy_didx(idxc1)

        @pl.when(not_last)
        def _():
            start_idx(2 * p + 3, idxc1, semi1)
        compute_scatter(rows1)
        return 0
    lax.fori_loop(0, npairs, pair, 0)

    plsc.subcore_barrier()
    pltpu.sync_copy(acc_sh.at[pl.ds(base, rows_per_sub)],
                    num_out.at[c].at[pl.ds(base, rows_per_sub)])
    pltpu.sync_copy(den_sh.at[pl.ds(base, rows_per_sub)],
                    den_out.at[c].at[pl.ds(base, rows_per_sub)])


def _edge_pass(tab, att, idxw):
    npad = tab.shape[0] // 2
    nchunks = idxw.shape[1]
    mesh = plsc.VectorSubcoreMesh(core_axis_name="c", subcore_axis_name="s")
    f = pl.kernel(
        functools.partial(_edge_pass_body, nchunks, npad),
        out_type=[
            jax.ShapeDtypeStruct((NCORE, npad, FDIM), jnp.float32),
            jax.ShapeDtypeStruct((NCORE, npad), jnp.float32),
        ],
        mesh=mesh,
        scratch_types=[
            pltpu.VMEM((2, 2 * CHUNK), jnp.int32),
            pltpu.VMEM((2, 2 * CHUNK), jnp.int32),
            pltpu.VMEM((2 * CHUNK, FDIM // 2), jnp.int32),
            pltpu.VMEM((2 * CHUNK, FDIM // 2), jnp.int32),
            pltpu.VMEM((CHUNK, FDIM), jnp.float32),
            pltpu.VMEM((1, CHUNK), jnp.int32),
            pltpu.VMEM((CHUNK,), jnp.float32),
            pltpu.VMEM((FDIM,), jnp.float32),
            pltpu.VMEM_SHARED((npad, FDIM), jnp.float32),
            pltpu.VMEM_SHARED((npad,), jnp.float32),
            pltpu.SemaphoreType.DMA,
            pltpu.SemaphoreType.DMA,
            pltpu.SemaphoreType.DMA,
            pltpu.SemaphoreType.DMA,
        ],
        compiler_params=pltpu.CompilerParams(needs_layout_passes=False,
                                             use_tc_tiling_on_sc=False),
    )
    return f(tab, att, idxw)


def _combine_body(npad, num_hbm, den_hbm, b_hbm, h_out,
                  na, nb, da, db, recbuf, hbuf, bbuf):
    c = lax.axis_index("c")
    s = lax.axis_index("s")
    w = s * NCORE + c
    rows_per_w = npad // NW
    rblk = 64
    pltpu.sync_copy(b_hbm, bbuf)
    bv = [bbuf[pl.ds(k * LANES, LANES)] for k in range(FB)]
    base = w * rows_per_w
    for t in range(rows_per_w // rblk):
        off = base + t * rblk
        pltpu.sync_copy(num_hbm.at[0].at[pl.ds(off, rblk)], na)
        pltpu.sync_copy(num_hbm.at[1].at[pl.ds(off, rblk)], nb)
        pltpu.sync_copy(den_hbm.at[0].at[pl.ds(off, rblk)], da)
        pltpu.sync_copy(den_hbm.at[1].at[pl.ds(off, rblk)], db)
        for g in range(rblk // LANES):
            dv = (da[pl.ds(g * LANES, LANES)] + db[pl.ds(g * LANES, LANES)]
                  + 1e-16)
            recbuf[pl.ds(g * LANES, LANES)] = 1.0 / dv

        def rbody(g, _):
            rv16 = recbuf[pl.ds(g * LANES, LANES)]
            for i in range(LANES):
                rv = jnp.full((LANES,), rv16[i], jnp.float32)
                r = g * LANES + i
                for k in range(FB):
                    v = (na[r, pl.ds(k * LANES, LANES)] +
                         nb[r, pl.ds(k * LANES, LANES)]) * rv + bv[k]
                    hbuf[r, pl.ds(k * LANES, LANES)] = jnp.maximum(v, 0.0)
            return 0
        lax.fori_loop(0, rblk // LANES, rbody, 0)
        pltpu.sync_copy(hbuf, h_out.at[pl.ds(off, rblk)])


def _combine(num, den, b):
    npad = num.shape[1]
    rblk = 64
    mesh = plsc.VectorSubcoreMesh(core_axis_name="c", subcore_axis_name="s")
    f = pl.kernel(
        functools.partial(_combine_body, npad),
        out_type=jax.ShapeDtypeStruct((npad, FDIM), jnp.float32),
        mesh=mesh,
        scratch_types=[
            pltpu.VMEM((rblk, FDIM), jnp.float32),
            pltpu.VMEM((rblk, FDIM), jnp.float32),
            pltpu.VMEM((rblk,), jnp.float32),
            pltpu.VMEM((rblk,), jnp.float32),
            pltpu.VMEM((rblk,), jnp.float32),
            pltpu.VMEM((rblk, FDIM), jnp.float32),
            pltpu.VMEM((FDIM,), jnp.float32),
        ],
        compiler_params=pltpu.CompilerParams(needs_layout_passes=False),
    )
    return f(num, den, b)


def kernel(x, edge_index, Wl1, Wr1, att1, b1, Wl2, Wr2, att2, b2):
    n = x.shape[0]
    e = edge_index.shape[1]
    npad = ((n + 1 + 2047) // 2048) * 2048
    etot = e + n
    nchunks = -(-etot // (NW * CHUNK))
    nchunks = ((nchunks + 1) // 2) * 2
    epad = NW * nchunks * CHUNK

    idt = edge_index.dtype
    loop = jnp.arange(n, dtype=idt)
    src = jnp.concatenate(
        [edge_index[0], loop, jnp.zeros((epad - etot,), idt)]).astype(jnp.int32)
    dst = jnp.concatenate(
        [edge_index[1], loop, jnp.full((epad - etot,), n, idt)]).astype(jnp.int32)
    srcw = src.reshape(NW, nchunks, CHUNK)
    dstw = dst.reshape(NW, nchunks, CHUNK)
    idxw = jnp.stack([jnp.concatenate([srcw, dstw + npad], axis=-1),
                      jnp.concatenate([dstw, dstw], axis=-1)], axis=2)

    def pack_tab(tab):
        tb = tab.astype(jnp.bfloat16).reshape(2 * npad, FDIM // 2, 2)
        return jax.lax.bitcast_convert_type(tb, jnp.int32)

    perm = jnp.asarray(_PERM)
    inv = jnp.asarray(_INV)
    x_pad = jnp.pad(x, ((0, npad - n), (0, 0)))
    tab1 = pack_tab(_mm2(x_pad, Wl1, Wr1))
    num1, den1 = _edge_pass(tab1, att1[perm], idxw)
    h1 = _combine(num1, den1, b1[perm])
    tab2 = pack_tab(_mm2(h1, Wl2[perm], Wr2[perm]))
    num2, den2 = _edge_pass(tab2, att2[perm], idxw)
    h2 = _combine(num2, den2, b2[perm])
    return h2[:n, inv]


# speedup vs baseline: 2.1117x; 2.1117x over previous
import functools

import jax
import jax.numpy as jnp
from jax import lax
from jax.experimental import pallas as pl
from jax.experimental.pallas import tpu as pltpu
from jax.experimental.pallas import tpu_sc as plsc

NCORE = 2
NSUB = 16
NW = NCORE * NSUB
LANES = 16
CHUNK = 64
FDIM = 128
FB = FDIM // LANES


def _mm2_body(x_ref, wa_ref, wb_ref, o_ref):
    xb = x_ref[...]
    o_ref[0] = jnp.dot(xb, wa_ref[...], preferred_element_type=jnp.float32)
    o_ref[1] = jnp.dot(xb, wb_ref[...], preferred_element_type=jnp.float32)


def _mm2(x, wa, wb):
    npad = x.shape[0]
    blk = 1024
    return pl.pallas_call(
        _mm2_body,
        grid=(npad // blk,),
        in_specs=[
            pl.BlockSpec((blk, FDIM), lambda i: (i, 0)),
            pl.BlockSpec((FDIM, FDIM), lambda i: (0, 0)),
            pl.BlockSpec((FDIM, FDIM), lambda i: (0, 0)),
        ],
        out_specs=pl.BlockSpec((2, blk, FDIM), lambda i: (0, i, 0)),
        out_shape=jax.ShapeDtypeStruct((2, npad, FDIM), jnp.float32),
    )(x, wa, wb)


def _edge_pass_body(nchunks, npad, tab_hbm, att_hbm, idx_hbm,
                    num_out, den_out,
                    idxc0, idxc1, rows0, rows1, didxs,
                    exbuf, attbuf, acc_sh, den_sh,
                    semg0, semg1, semi0, semi1):
    c = lax.axis_index("c")
    s = lax.axis_index("s")
    w = s * NCORE + c
    rows_per_sub = npad // NSUB
    lane = lax.iota(jnp.int32, LANES)
    zero16 = jnp.zeros((LANES,), jnp.float32)

    def zrow(e, _):
        for k in range(FB):
            rows0[e, pl.ds(k * LANES, LANES)] = zero16
        return 0
    lax.fori_loop(0, CHUNK, zrow, 0)
    for k in range(CHUNK // LANES):
        exbuf[pl.ds(k * LANES, LANES)] = zero16
    base = s * rows_per_sub
    for t in range(rows_per_sub // CHUNK):
        pltpu.sync_copy(rows0.at[pl.ds(0, CHUNK)],
                        acc_sh.at[pl.ds(base + t * CHUNK, CHUNK)])
        pltpu.sync_copy(exbuf, den_sh.at[pl.ds(base + t * CHUNK, CHUNK)])
    plsc.subcore_barrier()

    pltpu.sync_copy(att_hbm, attbuf)
    attv = [attbuf[pl.ds(k * LANES, LANES)] for k in range(FB)]

    def start_idx(j, idxc, semi):
        pltpu.async_copy(idx_hbm.at[w].at[j], idxc, semi)

    def wait_idx(idxc, semi):
        pltpu.make_async_copy(idx_hbm.at[w].at[0], idxc, semi).wait()

    def start_gather(idxc, rows, semg):
        pltpu.async_copy(tab_hbm.at[idxc.at[0]], rows, semg)

    def wait_gather(rows, semg):
        pltpu.make_async_copy(tab_hbm.at[pl.ds(0, 2 * CHUNK)], rows,
                              semg).wait()

    def compute_scatter(rows):
        @plsc.parallel_loop(0, CHUNK, unroll=2)
        def ebody(e):
            acc = zero16
            for k in range(FB):
                z = (rows[e, pl.ds(k * LANES, LANES)] +
                     rows[CHUNK + e, pl.ds(k * LANES, LANES)])
                acc = acc + attv[k] * jnp.maximum(z, 0.2 * z)
            rows[CHUNK + e, pl.ds(0, LANES)] = acc

        @plsc.parallel_loop(0, CHUNK // LANES, unroll=2)
        def bbody(g):
            elan = CHUNK + g * LANES + lane
            lsum = plsc.load_gather(rows,
                                    [elan, jnp.zeros((LANES,), jnp.int32)])
            for i in range(1, LANES):
                lsum = lsum + plsc.load_gather(
                    rows, [elan, jnp.full((LANES,), i, jnp.int32)])
            exbuf[pl.ds(g * LANES, LANES)] = jnp.exp(lsum)

        @plsc.parallel_loop(0, CHUNK // LANES)
        def cbody(g):
            ev = exbuf[pl.ds(g * LANES, LANES)]
            for i in range(LANES):
                exv = jnp.full((LANES,), ev[i], jnp.float32)
                e = g * LANES + i
                for k in range(FB):
                    rows[CHUNK + e, pl.ds(k * LANES, LANES)] = (
                        exv * rows[e, pl.ds(k * LANES, LANES)])

        pltpu.sync_copy(rows.at[pl.ds(CHUNK, CHUNK)],
                        acc_sh.at[didxs.at[0]], add=True)
        pltpu.sync_copy(exbuf, den_sh.at[didxs.at[0]], add=True)

    def copy_didx(idxc):
        for k in range(CHUNK // LANES):
            didxs[0, pl.ds(k * LANES, LANES)] = idxc[1, pl.ds(k * LANES,
                                                              LANES)]

    npairs = nchunks // 2
    start_idx(0, idxc0, semi0)
    start_idx(1, idxc1, semi1)
    wait_idx(idxc0, semi0)
    start_gather(idxc0, rows0, semg0)

    def pair(p, _):
        not_last = p < npairs - 1
        wait_idx(idxc1, semi1)
        start_gather(idxc1, rows1, semg1)
        wait_gather(rows0, semg0)
        copy_didx(idxc0)

        @pl.when(not_last)
        def _():
            start_idx(2 * p + 2, idxc0, semi0)
        compute_scatter(rows0)

        @pl.when(not_last)
        def _():
            wait_idx(idxc0, semi0)
            start_gather(idxc0, rows0, semg0)
        wait_gather(rows1, semg1)
        copy_didx(idxc1)

        @pl.when(not_last)
        def _():
            start_idx(2 * p + 3, idxc1, semi1)
        compute_scatter(rows1)
        return 0
    lax.fori_loop(0, npairs, pair, 0)

    plsc.subcore_barrier()
    pltpu.sync_copy(acc_sh.at[pl.ds(base, rows_per_sub)],
                    num_out.at[c].at[pl.ds(base, rows_per_sub)])
    pltpu.sync_copy(den_sh.at[pl.ds(base, rows_per_sub)],
                    den_out.at[c].at[pl.ds(base, rows_per_sub)])


def _edge_pass(tab, att, idxw):
    npad = tab.shape[0] // 2
    nchunks = idxw.shape[1]
    mesh = plsc.VectorSubcoreMesh(core_axis_name="c", subcore_axis_name="s")
    f = pl.kernel(
        functools.partial(_edge_pass_body, nchunks, npad),
        out_type=[
            jax.ShapeDtypeStruct((NCORE, npad, FDIM), jnp.float32),
            jax.ShapeDtypeStruct((NCORE, npad), jnp.float32),
        ],
        mesh=mesh,
        scratch_types=[
            pltpu.VMEM((2, 2 * CHUNK), jnp.int32),
            pltpu.VMEM((2, 2 * CHUNK), jnp.int32),
            pltpu.VMEM((2 * CHUNK, FDIM), jnp.float32),
            pltpu.VMEM((2 * CHUNK, FDIM), jnp.float32),
            pltpu.VMEM((1, CHUNK), jnp.int32),
            pltpu.VMEM((CHUNK,), jnp.float32),
            pltpu.VMEM((FDIM,), jnp.float32),
            pltpu.VMEM_SHARED((npad, FDIM), jnp.float32),
            pltpu.VMEM_SHARED((npad,), jnp.float32),
            pltpu.SemaphoreType.DMA,
            pltpu.SemaphoreType.DMA,
            pltpu.SemaphoreType.DMA,
            pltpu.SemaphoreType.DMA,
        ],
        compiler_params=pltpu.CompilerParams(needs_layout_passes=False),
    )
    return f(tab, att, idxw)


def _combine_body(npad, num_hbm, den_hbm, b_hbm, h_out,
                  na, nb, da, db, recbuf, hbuf, bbuf):
    c = lax.axis_index("c")
    s = lax.axis_index("s")
    w = s * NCORE + c
    rows_per_w = npad // NW
    rblk = 64
    pltpu.sync_copy(b_hbm, bbuf)
    bv = [bbuf[pl.ds(k * LANES, LANES)] for k in range(FB)]
    base = w * rows_per_w
    for t in range(rows_per_w // rblk):
        off = base + t * rblk
        pltpu.sync_copy(num_hbm.at[0].at[pl.ds(off, rblk)], na)
        pltpu.sync_copy(num_hbm.at[1].at[pl.ds(off, rblk)], nb)
        pltpu.sync_copy(den_hbm.at[0].at[pl.ds(off, rblk)], da)
        pltpu.sync_copy(den_hbm.at[1].at[pl.ds(off, rblk)], db)
        for g in range(rblk // LANES):
            dv = (da[pl.ds(g * LANES, LANES)] + db[pl.ds(g * LANES, LANES)]
                  + 1e-16)
            recbuf[pl.ds(g * LANES, LANES)] = 1.0 / dv

        def rbody(g, _):
            rv16 = recbuf[pl.ds(g * LANES, LANES)]
            for i in range(LANES):
                rv = jnp.full((LANES,), rv16[i], jnp.float32)
                r = g * LANES + i
                for k in range(FB):
                    v = (na[r, pl.ds(k * LANES, LANES)] +
                         nb[r, pl.ds(k * LANES, LANES)]) * rv + bv[k]
                    hbuf[r, pl.ds(k * LANES, LANES)] = jnp.maximum(v, 0.0)
            return 0
        lax.fori_loop(0, rblk // LANES, rbody, 0)
        pltpu.sync_copy(hbuf, h_out.at[pl.ds(off, rblk)])


def _combine(num, den, b):
    npad = num.shape[1]
    rblk = 64
    mesh = plsc.VectorSubcoreMesh(core_axis_name="c", subcore_axis_name="s")
    f = pl.kernel(
        functools.partial(_combine_body, npad),
        out_type=jax.ShapeDtypeStruct((npad, FDIM), jnp.float32),
        mesh=mesh,
        scratch_types=[
            pltpu.VMEM((rblk, FDIM), jnp.float32),
            pltpu.VMEM((rblk, FDIM), jnp.float32),
            pltpu.VMEM((rblk,), jnp.float32),
            pltpu.VMEM((rblk,), jnp.float32),
            pltpu.VMEM((rblk,), jnp.float32),
            pltpu.VMEM((rblk, FDIM), jnp.float32),
            pltpu.VMEM((FDIM,), jnp.float32),
        ],
        compiler_params=pltpu.CompilerParams(needs_layout_passes=False),
    )
    return f(num, den, b)


def kernel(x, edge_index, Wl1, Wr1, att1, b1, Wl2, Wr2, att2, b2):
    n = x.shape[0]
    e = edge_index.shape[1]
    npad = ((n + 1 + 2047) // 2048) * 2048
    etot = e + n
    nchunks = -(-etot // (NW * CHUNK))
    nchunks = ((nchunks + 1) // 2) * 2
    epad = NW * nchunks * CHUNK

    idt = edge_index.dtype
    loop = jnp.arange(n, dtype=idt)
    src = jnp.concatenate(
        [edge_index[0], loop, jnp.zeros((epad - etot,), idt)]).astype(jnp.int32)
    dst = jnp.concatenate(
        [edge_index[1], loop, jnp.full((epad - etot,), n, idt)]).astype(jnp.int32)
    srcw = src.reshape(NW, nchunks, CHUNK)
    dstw = dst.reshape(NW, nchunks, CHUNK)
    idxw = jnp.stack([jnp.concatenate([srcw, dstw + npad], axis=-1),
                      jnp.concatenate([dstw, dstw], axis=-1)], axis=2)

    x_pad = jnp.pad(x, ((0, npad - n), (0, 0)))
    tab1 = _mm2(x_pad, Wl1, Wr1).reshape(2 * npad, FDIM)
    num1, den1 = _edge_pass(tab1, att1, idxw)
    h1 = _combine(num1, den1, b1)
    tab2 = _mm2(h1, Wl2, Wr2).reshape(2 * npad, FDIM)
    num2, den2 = _edge_pass(tab2, att2, idxw)
    h2 = _combine(num2, den2, b2)
    return h2[:n]
